# baseline (speedup 1.0000x reference)
import functools

import jax
import jax.numpy as jnp
from jax import lax
from jax.experimental import pallas as pl
from jax.experimental.pallas import tpu as pltpu
from jax.experimental.pallas import tpu_sc as plsc

N = 100000
E = 1600000
T = 4
C_IN = 3
C_OUT = 16
K = 3
NUM_CLASSES = 10

NC = 2
NS = 16
NW = NC * NS

CH = 256
KCH = 5
MACRO = CH * KCH
ME = E // MACRO
MPW = ME // NW
MEXTRA = ME - MPW * NW

_mesh = plsc.VectorSubcoreMesh(core_axis_name="c", subcore_axis_name="s")
_sc_params = pltpu.CompilerParams(use_tc_tiling_on_sc=False)


def _worker_id():
    return lax.axis_index("s") * NC + lax.axis_index("c")


def _macro_range(wid):
    nm = MPW + jnp.where(wid < MEXTRA, 1, 0)
    m0 = wid * MPW + jnp.minimum(wid, MEXTRA)
    return m0, nm


@functools.partial(
    pl.kernel,
    mesh=_mesh,
    out_type=jax.ShapeDtypeStruct((NC, N), jnp.float32),
    scratch_types=[
        pltpu.VMEM((CH,), jnp.float32),
        pltpu.VMEM((800,), jnp.float32),
        [[pltpu.VMEM((CH,), jnp.int32) for _ in range(KCH)]
         for _ in range(2)],
        pltpu.VMEM_SHARED((N,), jnp.float32),
        pltpu.SemaphoreType.DMA,
        pltpu.SemaphoreType.DMA,
    ],
    compiler_params=_sc_params,
)
def _deg_call(rows_hbm, out_hbm, ones_v, z_v, idx_vs, dacc, s_idx, s_sc):
    cid = lax.axis_index("c")
    sid = lax.axis_index("s")
    wid = _worker_id()
    for i in range(CH // 16):
        ones_v[pl.ds(i * 16, 16)] = jnp.ones((16,), jnp.float32)
    for i in range(800 // 16):
        z_v[pl.ds(i * 16, 16)] = jnp.zeros((16,), jnp.float32)
    nz = jnp.where(sid < NS - 1, 8, 5)
    for k in range(8):
        @pl.when(k < nz)
        def _():
            pltpu.sync_copy(z_v, dacc.at[pl.ds(sid * 6400 + k * 800, 800)])

    m0, nm = _macro_range(wid)

    def fire_idx(t, m):
        e0 = m * MACRO
        for j in range(KCH):
            pltpu.async_copy(rows_hbm.at[pl.ds(e0 + j * CH, CH)],
                             idx_vs[t][j], s_idx)

    def wait_idx(t, m):
        e0 = m * MACRO
        for j in range(KCH):
            pltpu.make_async_copy(rows_hbm.at[pl.ds(e0 + j * CH, CH)],
                                  idx_vs[t][j], s_idx).wait()

    fire_idx(0, m0)
    plsc.subcore_barrier()

    def body(i, carry):
        m = m0 + i
        for t in (0, 1):
            @pl.when((i % 2) == t)
            def _(t=t):
                wait_idx(t, m)

                @pl.when(i + 1 < nm)
                def _():
                    fire_idx(1 - t, m + 1)
                copies = [pltpu.async_copy(ones_v, dacc.at[idx_vs[t][j]],
                                           s_sc, add=True)
                          for j in range(KCH)]
                for c in copies:
                    c.wait()
        return carry

    lax.fori_loop(0, nm, body, 0)
    plsc.subcore_barrier()
    for k in range(8):
        @pl.when(k < nz)
        def _():
            s0 = sid * 6400 + k * 800
            pltpu.sync_copy(dacc.at[pl.ds(s0, 800)],
                            out_hbm.at[cid, pl.ds(s0, 800)])


def _make_edge_pass(P):
    @functools.partial(
        pl.kernel,
        mesh=_mesh,
        out_type=(jax.ShapeDtypeStruct((NC, N, 16), jnp.float32),
                  jax.ShapeDtypeStruct((NC, N, 16), jnp.float32)),
        scratch_types=[
            [[pltpu.VMEM((CH,), jnp.int32) for _ in range(KCH)]
             for _ in range(2)],
            [[pltpu.VMEM((CH,), jnp.int32) for _ in range(KCH)]
             for _ in range(2)],
            [pltpu.VMEM((CH, 16), jnp.float32) for _ in range(KCH)],
            pltpu.VMEM((250, 16), jnp.float32),
            pltpu.VMEM_SHARED((N, 16), jnp.float32),
            pltpu.SemaphoreType.DMA,
            pltpu.SemaphoreType.DMA,
            pltpu.SemaphoreType.DMA,
        ],
        compiler_params=_sc_params,
    )
    def _edge_pass_call(tbl_hbm, scale_hbm, cols_hbm, rows_hbm,
                        out_hbm, tscr_hbm,
                        col_vs, row_vs, msg_vs, z_v, acc,
                        s_idx, s_g, s_sc):
        cid = lax.axis_index("c")
        sid = lax.axis_index("s")
        wid = _worker_id()
        for i in range(250):
            z_v[i, :] = jnp.zeros((16,), jnp.float32)
        for k in range(25):
            pltpu.sync_copy(z_v, acc.at[pl.ds(sid * 6250 + k * 250, 250)])

        base = sid * 6250
        for k in range(25):
            r0 = base + k * 250
            if P == 1:
                pltpu.sync_copy(tbl_hbm.at[pl.ds(r0, 250)],
                                msg_vs[0].at[pl.ds(0, 250)])
            else:
                pltpu.sync_copy(tbl_hbm.at[0, pl.ds(r0, 250)],
                                msg_vs[0].at[pl.ds(0, 250)])
                pltpu.sync_copy(tbl_hbm.at[1, pl.ds(r0, 250)],
                                msg_vs[1].at[pl.ds(0, 250)])
            pltpu.sync_copy(scale_hbm.at[pl.ds(r0, 250)],
                            msg_vs[P].at[pl.ds(0, 250)])

            def rowloop(r, carry):
                v = msg_vs[0][r, :]
                if P == 2:
                    v = v + msg_vs[1][r, :]
                msg_vs[0][r, :] = v * msg_vs[P][r, :]
                return carry
            lax.fori_loop(0, 250, rowloop, 0)
            pltpu.sync_copy(msg_vs[0].at[pl.ds(0, 250)],
                            tscr_hbm.at[cid, pl.ds(r0, 250)])

        m0, nm = _macro_range(wid)

        def fire_idx(t, m):
            e0 = m * MACRO
            for j in range(KCH):
                pltpu.async_copy(cols_hbm.at[pl.ds(e0 + j * CH, CH)],
                                 col_vs[t][j], s_idx)
                pltpu.async_copy(rows_hbm.at[pl.ds(e0 + j * CH, CH)],
                                 row_vs[t][j], s_idx)

        def wait_idx(t, m):
            e0 = m * MACRO
            for j in range(KCH):
                pltpu.make_async_copy(cols_hbm.at[pl.ds(e0 + j * CH, CH)],
                                      col_vs[t][j], s_idx).wait()
                pltpu.make_async_copy(rows_hbm.at[pl.ds(e0 + j * CH, CH)],
                                      row_vs[t][j], s_idx).wait()

        fire_idx(0, m0)
        mytbl = tscr_hbm.at[cid]
        plsc.subcore_barrier()

        def body(i, carry):
            m = m0 + i
            for t in (0, 1):
                @pl.when((i % 2) == t)
                def _(t=t):
                    wait_idx(t, m)
                    gathers = [pltpu.async_copy(mytbl.at[col_vs[t][j]],
                                                msg_vs[j], s_g)
                               for j in range(KCH)]

                    @pl.when(i + 1 < nm)
                    def _():
                        fire_idx(1 - t, m + 1)
                    for g in gathers:
                        g.wait()
                    scatters = [pltpu.async_copy(msg_vs[j],
                                                 acc.at[row_vs[t][j]],
                                                 s_sc, add=True)
                                for j in range(KCH)]
                    for c in scatters:
                        c.wait()
            return carry

        lax.fori_loop(0, nm, body, 0)
        plsc.subcore_barrier()
        for k in range(5):
            s0 = sid * 6250 + k * 1250
            pltpu.sync_copy(acc.at[pl.ds(s0, 1250)],
                            out_hbm.at[cid, pl.ds(s0, 1250)])

    return _edge_pass_call


_edge_pass1_call = _make_edge_pass(1)
_edge_pass2_call = _make_edge_pass(2)


NR = N * 16 // 128
BR = 512
G1 = -(-NR // BR)


def _tail1_body(x_ref, y1_ref, y2_ref, s_ref, w0_ref, w1_ref, w2_ref,
                bt_ref, o3_ref):
    s = s_ref[...]
    tx1 = -s * (y1_ref[0] + y1_ref[1])
    tx2 = -2.0 * s * (y2_ref[0] + y2_ref[1]) - x_ref[...]
    h = jnp.dot(x_ref[...], w0_ref[...], preferred_element_type=jnp.float32)
    h = h + jnp.dot(tx1, w1_ref[...], preferred_element_type=jnp.float32)
    h = h + jnp.dot(tx2, w2_ref[...], preferred_element_type=jnp.float32)
    h = jnp.maximum(h + bt_ref[...], 0.0)
    o3_ref[...] = (h[:, 0:128] + h[:, 128:256]
                   + h[:, 256:384] + h[:, 384:512])


_tail1_call = pl.pallas_call(
    _tail1_body,
    grid=(G1,),
    in_specs=[
        pl.BlockSpec((BR, 128), lambda i: (i, 0)),
        pl.BlockSpec((2, BR, 128), lambda i: (0, i, 0)),
        pl.BlockSpec((2, BR, 128), lambda i: (0, i, 0)),
        pl.BlockSpec((BR, 128), lambda i: (i, 0)),
        pl.BlockSpec((128, 512), lambda i: (0, 0)),
        pl.BlockSpec((128, 512), lambda i: (0, 0)),
        pl.BlockSpec((128, 512), lambda i: (0, 0)),
        pl.BlockSpec((1, 512), lambda i: (0, 0)),
    ],
    out_specs=pl.BlockSpec((BR, 128), lambda i: (i, 0)),
    out_shape=jax.ShapeDtypeStruct((NR, 128), jnp.float32),
)


QF = N * C_OUT // 4
BNF4 = 16000
G2 = QF // BNF4


def _tail2_body(fc0_ref, fc1_ref, fc2_ref, fc3_ref, v_ref, fcb_ref, out_ref):
    i = pl.program_id(0)

    @pl.when(i == 0)
    def _():
        out_ref[...] = jnp.zeros_like(out_ref)
        out_ref[0:NUM_CLASSES, 0:1] = fcb_ref[...]

    p = jnp.sum(fc0_ref[...] * v_ref[0:1, :], axis=1, keepdims=True)
    p += jnp.sum(fc1_ref[...] * v_ref[1:2, :], axis=1, keepdims=True)
    p += jnp.sum(fc2_ref[...] * v_ref[2:3, :], axis=1, keepdims=True)
    p += jnp.sum(fc3_ref[...] * v_ref[3:4, :], axis=1, keepdims=True)
    out_ref[0:NUM_CLASSES, 0:1] += p

    @pl.when(i == G2 - 1)
    def _():
        v = out_ref[0:NUM_CLASSES, 0:1]
        mx = jnp.max(v)
        ex = jnp.exp(v - mx)
        out_ref[0:NUM_CLASSES, 0:1] = v - mx - jnp.log(jnp.sum(ex))


_tail2_call = pl.pallas_call(
    _tail2_body,
    grid=(G2,),
    in_specs=[
        pl.BlockSpec((NUM_CLASSES, BNF4), lambda i: (0, i)),
        pl.BlockSpec((NUM_CLASSES, BNF4), lambda i: (0, i + G2)),
        pl.BlockSpec((NUM_CLASSES, BNF4), lambda i: (0, i + 2 * G2)),
        pl.BlockSpec((NUM_CLASSES, BNF4), lambda i: (0, i + 3 * G2)),
        pl.BlockSpec((4, BNF4), lambda i: (0, i)),
        pl.BlockSpec((NUM_CLASSES, 1), lambda i: (0, 0)),
    ],
    out_specs=pl.BlockSpec((16, 128), lambda i: (0, 0)),
    out_shape=jax.ShapeDtypeStruct((16, 128), jnp.float32),
)


def kernel(x, edge_index, W, b, hker, fc_w, fc_b):
    row = edge_index[0]
    col = edge_index[1]

    degp = _deg_call(row)
    deg = degp[0] + degp[1]
    dinv = jnp.where(deg > 0, lax.rsqrt(jnp.maximum(deg, 1e-12)), 0.0)

    x16 = jnp.pad(x.reshape(N, 12), ((0, 0), (0, 4)))
    x16p = x16.reshape(NR, 128)
    dinvp = jnp.broadcast_to(dinv[:, None], (N, 16)).reshape(NR, 128)

    dinv16 = dinvp.reshape(N, 16)
    negdsq16 = (-dinvp * dinvp).reshape(N, 16)
    y1, _ = _edge_pass1_call(x16, dinv16, col, row)
    y2, _ = _edge_pass2_call(y1, negdsq16, col, row)

    y1p = y1.reshape(2, NR, 128)
    y2p = y2.reshape(2, NR, 128)

    idx_t = (jnp.arange(T)[:, None] - jnp.arange(T)[None, :]) % T
    hmat = hker[idx_t]
    wbig = jnp.einsum('ts,kco->kscto', hmat, W).reshape(K * T * C_IN,
                                                        T * C_OUT)
    eye8 = jnp.eye(8, dtype=jnp.float32)
    wexps = []
    for k in range(K):
        wk = jnp.pad(wbig[k * 12:(k + 1) * 12, :], ((0, 4), (0, 0)))
        wkT = wk.reshape(16, T, C_OUT)
        wexps.append(jnp.einsum('gh,mto->gmtho', eye8, wkT).reshape(128, 512))
    bt512 = jnp.tile(b, 32).reshape(1, 512)

    o3p = _tail1_call(x16p, y1p, y2p, dinvp,
                      wexps[0], wexps[1], wexps[2], bt512)
    flat4 = o3p.reshape(4, QF)
    fcw = fc_w.reshape(NUM_CLASSES, N * C_OUT)
    logits_pad = _tail2_call(fcw, fcw, fcw, fcw, flat4,
                             fc_b.reshape(NUM_CLASSES, 1))
    return logits_pad[0:NUM_CLASSES, 0]


# speedup vs baseline: 1.1136x; 1.1136x over previous
import functools

import jax
import jax.numpy as jnp
from jax import lax
from jax.experimental import pallas as pl
from jax.experimental.pallas import tpu as pltpu
from jax.experimental.pallas import tpu_sc as plsc

N = 100000
E = 1600000
T = 4
C_IN = 3
C_OUT = 16
K = 3
NUM_CLASSES = 10

NC = 2
NS = 16
NW = NC * NS

CH = 128
KCH = 5
MACRO = CH * KCH
ME = E // MACRO
MPW = ME // NW
MEXTRA = ME - MPW * NW

_mesh = plsc.VectorSubcoreMesh(core_axis_name="c", subcore_axis_name="s")
_sc_params = pltpu.CompilerParams(use_tc_tiling_on_sc=False)


def _worker_id():
    return lax.axis_index("s") * NC + lax.axis_index("c")


def _macro_range(wid):
    nm = MPW + jnp.where(wid < MEXTRA, 1, 0)
    m0 = wid * MPW + jnp.minimum(wid, MEXTRA)
    return m0, nm


@functools.partial(
    pl.kernel,
    mesh=_mesh,
    out_type=jax.ShapeDtypeStruct((NC, N), jnp.float32),
    scratch_types=[
        pltpu.VMEM((CH,), jnp.float32),
        pltpu.VMEM((800,), jnp.float32),
        [[pltpu.VMEM((CH,), jnp.int32) for _ in range(KCH)]
         for _ in range(2)],
        pltpu.VMEM_SHARED((N,), jnp.float32),
        pltpu.SemaphoreType.DMA,
        pltpu.SemaphoreType.DMA,
    ],
    compiler_params=_sc_params,
)
def _deg_call(rows_hbm, out_hbm, ones_v, z_v, idx_vs, dacc, s_idx, s_sc):
    cid = lax.axis_index("c")
    sid = lax.axis_index("s")
    wid = _worker_id()
    for i in range(CH // 16):
        ones_v[pl.ds(i * 16, 16)] = jnp.ones((16,), jnp.float32)
    for i in range(800 // 16):
        z_v[pl.ds(i * 16, 16)] = jnp.zeros((16,), jnp.float32)
    nz = jnp.where(sid < NS - 1, 8, 5)
    for k in range(8):
        @pl.when(k < nz)
        def _():
            pltpu.sync_copy(z_v, dacc.at[pl.ds(sid * 6400 + k * 800, 800)])

    m0, nm = _macro_range(wid)

    def fire_idx(t, m):
        e0 = m * MACRO
        for j in range(KCH):
            pltpu.async_copy(rows_hbm.at[pl.ds(e0 + j * CH, CH)],
                             idx_vs[t][j], s_idx)

    def wait_idx(t, m):
        e0 = m * MACRO
        for j in range(KCH):
            pltpu.make_async_copy(rows_hbm.at[pl.ds(e0 + j * CH, CH)],
                                  idx_vs[t][j], s_idx).wait()

    fire_idx(0, m0)
    plsc.subcore_barrier()

    def body(i, carry):
        m = m0 + i
        for t in (0, 1):
            @pl.when((i % 2) == t)
            def _(t=t):
                wait_idx(t, m)

                @pl.when(i + 1 < nm)
                def _():
                    fire_idx(1 - t, m + 1)
                copies = [pltpu.async_copy(ones_v, dacc.at[idx_vs[t][j]],
                                           s_sc, add=True)
                          for j in range(KCH)]
                for c in copies:
                    c.wait()
        return carry

    lax.fori_loop(0, nm, body, 0)
    plsc.subcore_barrier()
    for k in range(8):
        @pl.when(k < nz)
        def _():
            s0 = sid * 6400 + k * 800
            pltpu.sync_copy(dacc.at[pl.ds(s0, 800)],
                            out_hbm.at[cid, pl.ds(s0, 800)])


@functools.partial(
    pl.kernel,
    mesh=_mesh,
    out_type=jax.ShapeDtypeStruct((NC, N, 16), jnp.float32),
    scratch_types=[
        [[pltpu.VMEM((CH,), jnp.int32) for _ in range(KCH)]
         for _ in range(2)],
        [[pltpu.VMEM((CH,), jnp.int32) for _ in range(KCH)]
         for _ in range(2)],
        [[pltpu.VMEM((CH, 16), jnp.float32) for _ in range(KCH)]
         for _ in range(2)],
        pltpu.VMEM((250, 16), jnp.float32),
        pltpu.VMEM_SHARED((N, 16), jnp.float32),
        pltpu.SemaphoreType.DMA,
        pltpu.SemaphoreType.DMA,
        pltpu.SemaphoreType.DMA,
    ],
    compiler_params=_sc_params,
)
def _edge_pass_call(table_hbm, cols_hbm, rows_hbm, out_hbm,
                    col_vs, row_vs, msg_vs, z_v, acc, s_idx, s_g, s_sc):
    cid = lax.axis_index("c")
    sid = lax.axis_index("s")
    wid = _worker_id()
    for i in range(250):
        z_v[i, :] = jnp.zeros((16,), jnp.float32)
    for k in range(25):
        pltpu.sync_copy(z_v, acc.at[pl.ds(sid * 6250 + k * 250, 250)])

    m0, nm = _macro_range(wid)

    def fire_idx(t, m):
        e0 = m * MACRO
        for j in range(KCH):
            pltpu.async_copy(cols_hbm.at[pl.ds(e0 + j * CH, CH)],
                             col_vs[t][j], s_idx)
            pltpu.async_copy(rows_hbm.at[pl.ds(e0 + j * CH, CH)],
                             row_vs[t][j], s_idx)

    def wait_idx(t, m):
        e0 = m * MACRO
        for j in range(KCH):
            pltpu.make_async_copy(cols_hbm.at[pl.ds(e0 + j * CH, CH)],
                                  col_vs[t][j], s_idx).wait()
            pltpu.make_async_copy(rows_hbm.at[pl.ds(e0 + j * CH, CH)],
                                  row_vs[t][j], s_idx).wait()

    def drain_sc(t):
        for j in range(KCH):
            pltpu.make_async_copy(msg_vs[t][j], acc.at[row_vs[t][j]],
                                  s_sc).wait()

    fire_idx(0, m0)
    plsc.subcore_barrier()

    def body(i, carry):
        m = m0 + i
        for t in (0, 1):
            @pl.when((i % 2) == t)
            def _(t=t):
                wait_idx(t, m)

                @pl.when(i >= 2)
                def _():
                    drain_sc(t)
                gathers = [pltpu.async_copy(table_hbm.at[col_vs[t][j]],
                                            msg_vs[t][j], s_g)
                           for j in range(KCH)]

                @pl.when(i + 1 < nm)
                def _():
                    fire_idx(1 - t, m + 1)
                for g in gathers:
                    g.wait()
                for j in range(KCH):
                    pltpu.async_copy(msg_vs[t][j], acc.at[row_vs[t][j]],
                                     s_sc, add=True)
        return carry

    lax.fori_loop(0, nm, body, 0)
    drain_sc(0)
    drain_sc(1)
    plsc.subcore_barrier()
    for k in range(5):
        s0 = sid * 6250 + k * 1250
        pltpu.sync_copy(acc.at[pl.ds(s0, 1250)],
                        out_hbm.at[cid, pl.ds(s0, 1250)])


NR = N * 16 // 128
BR = 512
G1 = -(-NR // BR)


def _tail1_body(x_ref, y1_ref, y2_ref, s_ref, w0_ref, w1_ref, w2_ref,
                bt_ref, o3_ref):
    s = s_ref[...]
    tx1 = -s * (y1_ref[0] + y1_ref[1])
    tx2 = -2.0 * s * (y2_ref[0] + y2_ref[1]) - x_ref[...]
    h = jnp.dot(x_ref[...], w0_ref[...], preferred_element_type=jnp.float32)
    h = h + jnp.dot(tx1, w1_ref[...], preferred_element_type=jnp.float32)
    h = h + jnp.dot(tx2, w2_ref[...], preferred_element_type=jnp.float32)
    h = jnp.maximum(h + bt_ref[...], 0.0)
    o3_ref[...] = (h[:, 0:128] + h[:, 128:256]
                   + h[:, 256:384] + h[:, 384:512])


_tail1_call = pl.pallas_call(
    _tail1_body,
    grid=(G1,),
    in_specs=[
        pl.BlockSpec((BR, 128), lambda i: (i, 0)),
        pl.BlockSpec((2, BR, 128), lambda i: (0, i, 0)),
        pl.BlockSpec((2, BR, 128), lambda i: (0, i, 0)),
        pl.BlockSpec((BR, 128), lambda i: (i, 0)),
        pl.BlockSpec((128, 512), lambda i: (0, 0)),
        pl.BlockSpec((128, 512), lambda i: (0, 0)),
        pl.BlockSpec((128, 512), lambda i: (0, 0)),
        pl.BlockSpec((1, 512), lambda i: (0, 0)),
    ],
    out_specs=pl.BlockSpec((BR, 128), lambda i: (i, 0)),
    out_shape=jax.ShapeDtypeStruct((NR, 128), jnp.float32),
)


QF = N * C_OUT // 4
BNF4 = 16000
G2 = QF // BNF4


def _tail2_body(fc0_ref, fc1_ref, fc2_ref, fc3_ref, v_ref, fcb_ref, out_ref):
    i = pl.program_id(0)

    @pl.when(i == 0)
    def _():
        out_ref[...] = jnp.zeros_like(out_ref)
        out_ref[0:NUM_CLASSES, 0:1] = fcb_ref[...]

    p = jnp.sum(fc0_ref[...] * v_ref[0:1, :], axis=1, keepdims=True)
    p += jnp.sum(fc1_ref[...] * v_ref[1:2, :], axis=1, keepdims=True)
    p += jnp.sum(fc2_ref[...] * v_ref[2:3, :], axis=1, keepdims=True)
    p += jnp.sum(fc3_ref[...] * v_ref[3:4, :], axis=1, keepdims=True)
    out_ref[0:NUM_CLASSES, 0:1] += p

    @pl.when(i == G2 - 1)
    def _():
        v = out_ref[0:NUM_CLASSES, 0:1]
        mx = jnp.max(v)
        ex = jnp.exp(v - mx)
        out_ref[0:NUM_CLASSES, 0:1] = v - mx - jnp.log(jnp.sum(ex))


_tail2_call = pl.pallas_call(
    _tail2_body,
    grid=(G2,),
    in_specs=[
        pl.BlockSpec((NUM_CLASSES, BNF4), lambda i: (0, i)),
        pl.BlockSpec((NUM_CLASSES, BNF4), lambda i: (0, i + G2)),
        pl.BlockSpec((NUM_CLASSES, BNF4), lambda i: (0, i + 2 * G2)),
        pl.BlockSpec((NUM_CLASSES, BNF4), lambda i: (0, i + 3 * G2)),
        pl.BlockSpec((4, BNF4), lambda i: (0, i)),
        pl.BlockSpec((NUM_CLASSES, 1), lambda i: (0, 0)),
    ],
    out_specs=pl.BlockSpec((16, 128), lambda i: (0, 0)),
    out_shape=jax.ShapeDtypeStruct((16, 128), jnp.float32),
)


def kernel(x, edge_index, W, b, hker, fc_w, fc_b):
    row = edge_index[0]
    col = edge_index[1]

    degp = _deg_call(row)
    deg = degp[0] + degp[1]
    dinv = jnp.where(deg > 0, lax.rsqrt(jnp.maximum(deg, 1e-12)), 0.0)

    x16 = jnp.pad(x.reshape(N, 12), ((0, 0), (0, 4)))
    x16p = x16.reshape(NR, 128)
    dinvp = jnp.broadcast_to(dinv[:, None], (N, 16)).reshape(NR, 128)

    t1 = x16 * dinv[:, None]
    y1 = _edge_pass_call(t1, col, row)

    t2 = -(dinv * dinv)[:, None] * (y1[0] + y1[1])
    y2 = _edge_pass_call(t2, col, row)

    y1p = y1.reshape(2, NR, 128)
    y2p = y2.reshape(2, NR, 128)

    idx_t = (jnp.arange(T)[:, None] - jnp.arange(T)[None, :]) % T
    hmat = hker[idx_t]
    wbig = jnp.einsum('ts,kco->kscto', hmat, W).reshape(K * T * C_IN,
                                                        T * C_OUT)
    eye8 = jnp.eye(8, dtype=jnp.float32)
    wexps = []
    for k in range(K):
        wk = jnp.pad(wbig[k * 12:(k + 1) * 12, :], ((0, 4), (0, 0)))
        wkT = wk.reshape(16, T, C_OUT)
        wexps.append(jnp.einsum('gh,mto->gmtho', eye8, wkT).reshape(128, 512))
    bt512 = jnp.tile(b, 32).reshape(1, 512)

    o3p = _tail1_call(x16p, y1p, y2p, dinvp,
                      wexps[0], wexps[1], wexps[2], bt512)
    flat4 = o3p.reshape(4, QF)
    fcw = fc_w.reshape(NUM_CLASSES, N * C_OUT)
    logits_pad = _tail2_call(fcw, fcw, fcw, fcw, flat4,
                             fc_b.reshape(NUM_CLASSES, 1))
    return logits_pad[0:NUM_CLASSES, 0]


# speedup vs baseline: 1.1415x; 1.0251x over previous
import functools

import jax
import jax.numpy as jnp
from jax import lax
from jax.experimental import pallas as pl
from jax.experimental.pallas import tpu as pltpu
from jax.experimental.pallas import tpu_sc as plsc

N = 100000
E = 1600000
T = 4
C_IN = 3
C_OUT = 16
K = 3
NUM_CLASSES = 10

NC = 2
NS = 16
NW = NC * NS

CH = 256
KCH = 5
MACRO = CH * KCH
ME = E // MACRO
MPW = ME // NW
MEXTRA = ME - MPW * NW

_mesh = plsc.VectorSubcoreMesh(core_axis_name="c", subcore_axis_name="s")
_sc_params = pltpu.CompilerParams(use_tc_tiling_on_sc=False)


def _worker_id():
    return lax.axis_index("s") * NC + lax.axis_index("c")


def _macro_range(wid):
    nm = MPW + jnp.where(wid < MEXTRA, 1, 0)
    m0 = wid * MPW + jnp.minimum(wid, MEXTRA)
    return m0, nm


@functools.partial(
    pl.kernel,
    mesh=_mesh,
    out_type=jax.ShapeDtypeStruct((NC, N), jnp.float32),
    scratch_types=[
        pltpu.VMEM((CH,), jnp.float32),
        pltpu.VMEM((800,), jnp.float32),
        [[pltpu.VMEM((CH,), jnp.int32) for _ in range(KCH)]
         for _ in range(2)],
        pltpu.VMEM_SHARED((N,), jnp.float32),
        pltpu.SemaphoreType.DMA,
        pltpu.SemaphoreType.DMA,
    ],
    compiler_params=_sc_params,
)
def _deg_call(rows_hbm, out_hbm, ones_v, z_v, idx_vs, dacc, s_idx, s_sc):
    cid = lax.axis_index("c")
    sid = lax.axis_index("s")
    wid = _worker_id()
    for i in range(CH // 16):
        ones_v[pl.ds(i * 16, 16)] = jnp.ones((16,), jnp.float32)
    for i in range(800 // 16):
        z_v[pl.ds(i * 16, 16)] = jnp.zeros((16,), jnp.float32)
    nz = jnp.where(sid < NS - 1, 8, 5)
    for k in range(8):
        @pl.when(k < nz)
        def _():
            pltpu.sync_copy(z_v, dacc.at[pl.ds(sid * 6400 + k * 800, 800)])

    m0, nm = _macro_range(wid)

    def fire_idx(t, m):
        e0 = m * MACRO
        for j in range(KCH):
            pltpu.async_copy(rows_hbm.at[pl.ds(e0 + j * CH, CH)],
                             idx_vs[t][j], s_idx)

    def wait_idx(t, m):
        e0 = m * MACRO
        for j in range(KCH):
            pltpu.make_async_copy(rows_hbm.at[pl.ds(e0 + j * CH, CH)],
                                  idx_vs[t][j], s_idx).wait()

    fire_idx(0, m0)
    plsc.subcore_barrier()

    def body(i, carry):
        m = m0 + i
        for t in (0, 1):
            @pl.when((i % 2) == t)
            def _(t=t):
                wait_idx(t, m)

                @pl.when(i + 1 < nm)
                def _():
                    fire_idx(1 - t, m + 1)
                copies = [pltpu.async_copy(ones_v, dacc.at[idx_vs[t][j]],
                                           s_sc, add=True)
                          for j in range(KCH)]
                for c in copies:
                    c.wait()
        return carry

    lax.fori_loop(0, nm, body, 0)
    plsc.subcore_barrier()
    for k in range(8):
        @pl.when(k < nz)
        def _():
            s0 = sid * 6400 + k * 800
            pltpu.sync_copy(dacc.at[pl.ds(s0, 800)],
                            out_hbm.at[cid, pl.ds(s0, 800)])


@functools.partial(
    pl.kernel,
    mesh=_mesh,
    out_type=jax.ShapeDtypeStruct((NC, N, 16), jnp.float32),
    scratch_types=[
        [[pltpu.VMEM((CH,), jnp.int32) for _ in range(KCH)]
         for _ in range(2)],
        [[pltpu.VMEM((CH,), jnp.int32) for _ in range(KCH)]
         for _ in range(2)],
        [pltpu.VMEM((CH, 16), jnp.float32) for _ in range(KCH)],
        pltpu.VMEM((250, 16), jnp.float32),
        pltpu.VMEM_SHARED((N, 16), jnp.float32),
        pltpu.SemaphoreType.DMA,
        pltpu.SemaphoreType.DMA,
        pltpu.SemaphoreType.DMA,
    ],
    compiler_params=_sc_params,
)
def _edge_pass_call(table_hbm, cols_hbm, rows_hbm, out_hbm,
                    col_vs, row_vs, msg_vs, z_v, acc, s_idx, s_g, s_sc):
    cid = lax.axis_index("c")
    sid = lax.axis_index("s")
    wid = _worker_id()
    for i in range(250):
        z_v[i, :] = jnp.zeros((16,), jnp.float32)
    for k in range(25):
        pltpu.sync_copy(z_v, acc.at[pl.ds(sid * 6250 + k * 250, 250)])

    m0, nm = _macro_range(wid)

    def fire_idx(t, m):
        e0 = m * MACRO
        for j in range(KCH):
            pltpu.async_copy(cols_hbm.at[pl.ds(e0 + j * CH, CH)],
                             col_vs[t][j], s_idx)
            pltpu.async_copy(rows_hbm.at[pl.ds(e0 + j * CH, CH)],
                             row_vs[t][j], s_idx)

    def wait_idx(t, m):
        e0 = m * MACRO
        for j in range(KCH):
            pltpu.make_async_copy(cols_hbm.at[pl.ds(e0 + j * CH, CH)],
                                  col_vs[t][j], s_idx).wait()
            pltpu.make_async_copy(rows_hbm.at[pl.ds(e0 + j * CH, CH)],
                                  row_vs[t][j], s_idx).wait()

    fire_idx(0, m0)
    plsc.subcore_barrier()

    def body(i, carry):
        m = m0 + i
        for t in (0, 1):
            @pl.when((i % 2) == t)
            def _(t=t):
                wait_idx(t, m)
                gathers = [pltpu.async_copy(table_hbm.at[col_vs[t][j]],
                                            msg_vs[j], s_g)
                           for j in range(KCH)]

                @pl.when(i + 1 < nm)
                def _():
                    fire_idx(1 - t, m + 1)
                for g in gathers:
                    g.wait()
                scatters = [pltpu.async_copy(msg_vs[j],
                                             acc.at[row_vs[t][j]],
                                             s_sc, add=True)
                            for j in range(KCH)]
                for c in scatters:
                    c.wait()
        return carry

    lax.fori_loop(0, nm, body, 0)
    plsc.subcore_barrier()
    for k in range(5):
        s0 = sid * 6250 + k * 1250
        pltpu.sync_copy(acc.at[pl.ds(s0, 1250)],
                        out_hbm.at[cid, pl.ds(s0, 1250)])


NR = N * 16 // 128
BR = 512
G1 = -(-NR // BR)


def _tail1_body(x_ref, y1_ref, y2_ref, s_ref, w0_ref, w1_ref, w2_ref,
                bt_ref, o3_ref):
    s = s_ref[...]
    tx1 = -s * (y1_ref[0] + y1_ref[1])
    tx2 = -2.0 * s * (y2_ref[0] + y2_ref[1]) - x_ref[...]
    h = jnp.dot(x_ref[...], w0_ref[...], preferred_element_type=jnp.float32)
    h = h + jnp.dot(tx1, w1_ref[...], preferred_element_type=jnp.float32)
    h = h + jnp.dot(tx2, w2_ref[...], preferred_element_type=jnp.float32)
    h = jnp.maximum(h + bt_ref[...], 0.0)
    o3_ref[...] = (h[:, 0:128] + h[:, 128:256]
                   + h[:, 256:384] + h[:, 384:512])


_tail1_call = pl.pallas_call(
    _tail1_body,
    grid=(G1,),
    in_specs=[
        pl.BlockSpec((BR, 128), lambda i: (i, 0)),
        pl.BlockSpec((2, BR, 128), lambda i: (0, i, 0)),
        pl.BlockSpec((2, BR, 128), lambda i: (0, i, 0)),
        pl.BlockSpec((BR, 128), lambda i: (i, 0)),
        pl.BlockSpec((128, 512), lambda i: (0, 0)),
        pl.BlockSpec((128, 512), lambda i: (0, 0)),
        pl.BlockSpec((128, 512), lambda i: (0, 0)),
        pl.BlockSpec((1, 512), lambda i: (0, 0)),
    ],
    out_specs=pl.BlockSpec((BR, 128), lambda i: (i, 0)),
    out_shape=jax.ShapeDtypeStruct((NR, 128), jnp.float32),
)


QF = N * C_OUT // 4
BNF4 = 16000
G2 = QF // BNF4


def _tail2_body(fc0_ref, fc1_ref, fc2_ref, fc3_ref, v_ref, fcb_ref, out_ref):
    i = pl.program_id(0)

    @pl.when(i == 0)
    def _():
        out_ref[...] = jnp.zeros_like(out_ref)
        out_ref[0:NUM_CLASSES, 0:1] = fcb_ref[...]

    p = jnp.sum(fc0_ref[...] * v_ref[0:1, :], axis=1, keepdims=True)
    p += jnp.sum(fc1_ref[...] * v_ref[1:2, :], axis=1, keepdims=True)
    p += jnp.sum(fc2_ref[...] * v_ref[2:3, :], axis=1, keepdims=True)
    p += jnp.sum(fc3_ref[...] * v_ref[3:4, :], axis=1, keepdims=True)
    out_ref[0:NUM_CLASSES, 0:1] += p

    @pl.when(i == G2 - 1)
    def _():
        v = out_ref[0:NUM_CLASSES, 0:1]
        mx = jnp.max(v)
        ex = jnp.exp(v - mx)
        out_ref[0:NUM_CLASSES, 0:1] = v - mx - jnp.log(jnp.sum(ex))


_tail2_call = pl.pallas_call(
    _tail2_body,
    grid=(G2,),
    in_specs=[
        pl.BlockSpec((NUM_CLASSES, BNF4), lambda i: (0, i)),
        pl.BlockSpec((NUM_CLASSES, BNF4), lambda i: (0, i + G2)),
        pl.BlockSpec((NUM_CLASSES, BNF4), lambda i: (0, i + 2 * G2)),
        pl.BlockSpec((NUM_CLASSES, BNF4), lambda i: (0, i + 3 * G2)),
        pl.BlockSpec((4, BNF4), lambda i: (0, i)),
        pl.BlockSpec((NUM_CLASSES, 1), lambda i: (0, 0)),
    ],
    out_specs=pl.BlockSpec((16, 128), lambda i: (0, 0)),
    out_shape=jax.ShapeDtypeStruct((16, 128), jnp.float32),
)


def kernel(x, edge_index, W, b, hker, fc_w, fc_b):
    row = edge_index[0]
    col = edge_index[1]

    degp = _deg_call(row)
    deg = degp[0] + degp[1]
    dinv = jnp.where(deg > 0, lax.rsqrt(jnp.maximum(deg, 1e-12)), 0.0)

    x16 = jnp.pad(x.reshape(N, 12), ((0, 0), (0, 4)))
    x16p = x16.reshape(NR, 128)
    dinvp = jnp.broadcast_to(dinv[:, None], (N, 16)).reshape(NR, 128)

    t1 = x16 * dinv[:, None]
    y1 = _edge_pass_call(t1, col, row)

    t2 = -(dinv * dinv)[:, None] * (y1[0] + y1[1])
    y2 = _edge_pass_call(t2, col, row)

    y1p = y1.reshape(2, NR, 128)
    y2p = y2.reshape(2, NR, 128)

    idx_t = (jnp.arange(T)[:, None] - jnp.arange(T)[None, :]) % T
    hmat = hker[idx_t]
    wbig = jnp.einsum('ts,kco->kscto', hmat, W).reshape(K * T * C_IN,
                                                        T * C_OUT)
    eye8 = jnp.eye(8, dtype=jnp.float32)
    wexps = []
    for k in range(K):
        wk = jnp.pad(wbig[k * 12:(k + 1) * 12, :], ((0, 4), (0, 0)))
        wkT = wk.reshape(16, T, C_OUT)
        wexps.append(jnp.einsum('gh,mto->gmtho', eye8, wkT).reshape(128, 512))
    bt512 = jnp.tile(b, 32).reshape(1, 512)

    o3p = _tail1_call(x16p, y1p, y2p, dinvp,
                      wexps[0], wexps[1], wexps[2], bt512)
    flat4 = o3p.reshape(4, QF)
    fcw = fc_w.reshape(NUM_CLASSES, N * C_OUT)
    logits_pad = _tail2_call(fcw, fcw, fcw, fcw, flat4,
                             fc_b.reshape(NUM_CLASSES, 1))
    return logits_pad[0:NUM_CLASSES, 0]
